# Initial kernel scaffold; baseline (speedup 1.0000x reference)
import functools

import jax
import jax.numpy as jnp
from jax import lax
from jax.experimental import pallas as pl
from jax.experimental.pallas import tpu as pltpu
from jax.experimental.pallas import tpu_sc as plsc

EMBED = 64
BATCH = 128
CHUNK = 512
NBATCH = CHUNK // BATCH


@functools.cache
def _build(b_total: int, vocab: int, embed: int):
    info = plsc.get_sparse_core_info()
    nw = info.num_cores * info.num_subcores
    b_per_w = b_total // nw
    nchunk = b_per_w // CHUNK
    mesh = plsc.VectorSubcoreMesh(core_axis_name="c", subcore_axis_name="s")

    @functools.partial(
        pl.kernel,
        mesh=mesh,
        out_type=jax.ShapeDtypeStruct((b_total, embed), jnp.float32),
        scratch_types=[
            pltpu.VMEM((b_per_w,), jnp.int32),
            pltpu.VMEM((CHUNK, embed), jnp.float32),
            pltpu.SemaphoreType.DMA,
            pltpu.SemaphoreType.DMA,
        ],
    )
    def emb(table_hbm, idx_hbm, out_hbm, idx_v, rows_v, sem_i, sem_g):
        wid = lax.axis_index("s") * info.num_cores + lax.axis_index("c")
        base = wid * b_per_w
        pltpu.async_copy(idx_hbm.at[pl.ds(base, b_per_w)], idx_v, sem_i).wait()

        def body(g, carry):
            off = g * CHUNK
            copies = []
            for j in range(NBATCH):
                copies.append(
                    pltpu.async_copy(
                        table_hbm.at[idx_v.at[pl.ds(off + j * BATCH, BATCH)]],
                        rows_v.at[pl.ds(j * BATCH, BATCH)],
                        sem_g,
                    )
                )
            for c in copies:
                c.wait()
            pltpu.sync_copy(rows_v, out_hbm.at[pl.ds(base + off, CHUNK)])
            return carry

        lax.fori_loop(0, nchunk, body, 0)

    return emb


def kernel(x, weight):
    b, s = x.shape
    idx = x.reshape(-1).astype(jnp.int32)
    out = _build(idx.shape[0], weight.shape[0], weight.shape[1])(weight, idx)
    return out.reshape(b, s, EMBED)


# speedup vs baseline: 1.8302x; 1.8302x over previous
import functools

import jax
import jax.numpy as jnp
from jax import lax
from jax.experimental import pallas as pl
from jax.experimental.pallas import tpu as pltpu
from jax.experimental.pallas import tpu_sc as plsc

EMBED = 64
BATCH = 128
CHUNK = 512
NBATCH = CHUNK // BATCH


@functools.cache
def _build(b_total: int, vocab: int, embed: int):
    info = plsc.get_sparse_core_info()
    nw = info.num_cores * info.num_subcores
    b_per_w = b_total // nw
    nchunk = b_per_w // CHUNK
    mesh = plsc.VectorSubcoreMesh(core_axis_name="c", subcore_axis_name="s")

    @functools.partial(
        pl.kernel,
        mesh=mesh,
        out_type=jax.ShapeDtypeStruct((b_total, embed), jnp.float32),
        scratch_types=[
            pltpu.VMEM((b_per_w,), jnp.int32),
            pltpu.VMEM((CHUNK, embed), jnp.float32),
            pltpu.SemaphoreType.DMA,
            pltpu.SemaphoreType.DMA,
        ],
        compiler_params=pltpu.CompilerParams(use_tc_tiling_on_sc=False),
    )
    def emb(table_hbm, idx_hbm, out_hbm, idx_v, rows_v, sem_i, sem_g):
        wid = lax.axis_index("s") * info.num_cores + lax.axis_index("c")
        base = wid * b_per_w
        pltpu.async_copy(idx_hbm.at[pl.ds(base, b_per_w)], idx_v, sem_i).wait()

        def body(g, carry):
            off = g * CHUNK
            copies = []
            for j in range(NBATCH):
                copies.append(
                    pltpu.async_copy(
                        table_hbm.at[idx_v.at[pl.ds(off + j * BATCH, BATCH)]],
                        rows_v.at[pl.ds(j * BATCH, BATCH)],
                        sem_g,
                    )
                )
            for c in copies:
                c.wait()
            pltpu.sync_copy(rows_v, out_hbm.at[pl.ds(base + off, CHUNK)])
            return carry

        lax.fori_loop(0, nchunk, body, 0)

    return emb


def kernel(x, weight):
    b, s = x.shape
    idx = x.reshape(-1).astype(jnp.int32)
    out = _build(idx.shape[0], weight.shape[0], weight.shape[1])(weight, idx)
    return out.reshape(b, s, EMBED)


# speedup vs baseline: 1.8316x; 1.0007x over previous
import functools

import jax
import jax.numpy as jnp
from jax import lax
from jax.experimental import pallas as pl
from jax.experimental.pallas import tpu as pltpu
from jax.experimental.pallas import tpu_sc as plsc

EMBED = 64
BATCH = 512
CHUNK = 512
NBATCH = CHUNK // BATCH


@functools.cache
def _build(b_total: int, vocab: int, embed: int):
    info = plsc.get_sparse_core_info()
    nw = info.num_cores * info.num_subcores
    b_per_w = b_total // nw
    nchunk = b_per_w // CHUNK
    mesh = plsc.VectorSubcoreMesh(core_axis_name="c", subcore_axis_name="s")

    @functools.partial(
        pl.kernel,
        mesh=mesh,
        out_type=jax.ShapeDtypeStruct((b_total, embed), jnp.float32),
        scratch_types=[
            pltpu.VMEM((b_per_w,), jnp.int32),
            pltpu.VMEM((CHUNK, embed), jnp.float32),
            pltpu.SemaphoreType.DMA,
            pltpu.SemaphoreType.DMA,
        ],
        compiler_params=pltpu.CompilerParams(use_tc_tiling_on_sc=False),
    )
    def emb(table_hbm, idx_hbm, out_hbm, idx_v, rows_v, sem_i, sem_g):
        wid = lax.axis_index("s") * info.num_cores + lax.axis_index("c")
        base = wid * b_per_w
        pltpu.async_copy(idx_hbm.at[pl.ds(base, b_per_w)], idx_v, sem_i).wait()

        def body(g, carry):
            off = g * CHUNK
            copies = []
            for j in range(NBATCH):
                copies.append(
                    pltpu.async_copy(
                        table_hbm.at[idx_v.at[pl.ds(off + j * BATCH, BATCH)]],
                        rows_v.at[pl.ds(j * BATCH, BATCH)],
                        sem_g,
                    )
                )
            for c in copies:
                c.wait()
            pltpu.sync_copy(rows_v, out_hbm.at[pl.ds(base + off, CHUNK)])
            return carry

        lax.fori_loop(0, nchunk, body, 0)

    return emb


def kernel(x, weight):
    b, s = x.shape
    idx = x.reshape(-1).astype(jnp.int32)
    out = _build(idx.shape[0], weight.shape[0], weight.shape[1])(weight, idx)
    return out.reshape(b, s, EMBED)


# speedup vs baseline: 1.8746x; 1.0235x over previous
import functools

import jax
import jax.numpy as jnp
from jax import lax
from jax.experimental import pallas as pl
from jax.experimental.pallas import tpu as pltpu
from jax.experimental.pallas import tpu_sc as plsc

EMBED = 64
BATCH = 512
CHUNK = 512
NBATCH = CHUNK // BATCH


@functools.cache
def _build(b_total: int, vocab: int, embed: int):
    info = plsc.get_sparse_core_info()
    nw = info.num_cores * info.num_subcores
    b_per_w = b_total // nw
    nchunk = b_per_w // CHUNK
    mesh = plsc.VectorSubcoreMesh(core_axis_name="c", subcore_axis_name="s")

    npair = nchunk // 2

    @functools.partial(
        pl.kernel,
        mesh=mesh,
        out_type=jax.ShapeDtypeStruct((b_total, embed), jnp.float32),
        scratch_types=[
            pltpu.VMEM((b_per_w,), jnp.int32),
            pltpu.VMEM((2, CHUNK, embed), jnp.float32),
            pltpu.SemaphoreType.DMA,
            pltpu.SemaphoreType.DMA,
            pltpu.SemaphoreType.DMA,
            pltpu.SemaphoreType.DMA,
            pltpu.SemaphoreType.DMA,
        ],
        compiler_params=pltpu.CompilerParams(use_tc_tiling_on_sc=False),
    )
    def emb(table_hbm, idx_hbm, out_hbm, idx_v, rows_v, sem_i, sem_g0, sem_g1,
            sem_w0, sem_w1):
        wid = lax.axis_index("s") * info.num_cores + lax.axis_index("c")
        base = wid * b_per_w
        pltpu.async_copy(idx_hbm.at[pl.ds(base, b_per_w)], idx_v, sem_i).wait()
        sem_g = (sem_g0, sem_g1)
        sem_w = (sem_w0, sem_w1)

        def gather_desc(g, slot):
            return pltpu.make_async_copy(
                table_hbm.at[idx_v.at[pl.ds(g * CHUNK, CHUNK)]],
                rows_v.at[slot],
                sem_g[slot],
            )

        def write_desc(g, slot):
            return pltpu.make_async_copy(
                rows_v.at[slot],
                out_hbm.at[pl.ds(base + g * CHUNK, CHUNK)],
                sem_w[slot],
            )

        gather_desc(0, 0).start()

        def body(i, carry):
            g = i * 2
            @pl.when(i > 0)
            def _():
                write_desc(g - 1, 1).wait()

            gather_desc(g + 1, 1).start()
            gather_desc(g, 0).wait()
            write_desc(g, 0).start()
            write_desc(g, 0).wait()

            @pl.when(i + 1 < npair)
            def _():
                gather_desc(g + 2, 0).start()

            gather_desc(g + 1, 1).wait()
            write_desc(g + 1, 1).start()
            return carry

        lax.fori_loop(0, npair, body, 0)
        write_desc(nchunk - 1, 1).wait()

    return emb


def kernel(x, weight):
    b, s = x.shape
    idx = x.reshape(-1).astype(jnp.int32)
    out = _build(idx.shape[0], weight.shape[0], weight.shape[1])(weight, idx)
    return out.reshape(b, s, EMBED)
